# Initial kernel scaffold; baseline (speedup 1.0000x reference)
import jax
import jax.numpy as jnp
from jax.experimental import pallas as pl


def kernel(centers, context, neg_context, in_emb, out_emb):
    raise NotImplementedError("write your pallas kernel here")


# speedup vs baseline: 8.5886x; 8.5886x over previous
import functools

import jax
import jax.numpy as jnp
from jax import lax
from jax.experimental import pallas as pl
from jax.experimental.pallas import tpu as pltpu
from jax.experimental.pallas import tpu_sc as plsc

VOCAB = 100000
D = 128
B = 16384
NCTX = 4
NNEG = 5
NIDX = 1 + NCTX + NNEG

NC = 2
NS = 16
NW = NC * NS
BPW = B // NW
K = 128
NCHUNK = BPW // K


def _sc_body(in_emb, out_emb, idxb, pos_hbm, neg_hbm,
             idx_bufs, u_v, vsum_v, nsum_v, pos_v, neg_v, sem):
    cid = lax.axis_index("c")
    sid = lax.axis_index("s")
    wid = sid * NC + cid
    for t in range(NCHUNK):
        m = wid * NCHUNK + t
        base = m * K
        idx_cps = [pltpu.async_copy(idxb.at[m, r], idx_bufs[r], sem)
                   for r in range(NIDX)]
        for cp in idx_cps:
            cp.wait()
        cp_u = pltpu.async_copy(in_emb.at[idx_bufs[0]], u_v, sem)
        cp_v = pltpu.async_copy(out_emb.at[idx_bufs[1]], vsum_v, sem)
        cp_n = pltpu.async_copy(out_emb.at[idx_bufs[1 + NCTX]], nsum_v, sem)
        cp_u.wait()
        cp_v.wait()
        cp_n.wait()
        adds = []
        for r in range(2, 1 + NCTX):
            adds.append(pltpu.async_copy(out_emb.at[idx_bufs[r]], vsum_v, sem,
                                         add=True))
        for r in range(2 + NCTX, NIDX):
            adds.append(pltpu.async_copy(out_emb.at[idx_bufs[r]], nsum_v, sem,
                                         add=True))
        for cp in adds:
            cp.wait()

        def elem(k, carry):
            accp = u_v[k, pl.ds(0, 16)] * vsum_v[k, pl.ds(0, 16)]
            accn = u_v[k, pl.ds(0, 16)] * nsum_v[k, pl.ds(0, 16)]
            for j in range(1, 8):
                uu = u_v[k, pl.ds(16 * j, 16)]
                accp = accp + uu * vsum_v[k, pl.ds(16 * j, 16)]
                accn = accn + uu * nsum_v[k, pl.ds(16 * j, 16)]
            pos_v[k, pl.ds(0, 16)] = accp
            neg_v[k, pl.ds(0, 16)] = accn
            return carry

        lax.fori_loop(0, K, elem, 0)
        pltpu.sync_copy(pos_v, pos_hbm.at[pl.ds(base, K)])
        pltpu.sync_copy(neg_v, neg_hbm.at[pl.ds(base, K)])


def _scores_sc(in_emb, out_emb, idxb):
    mesh = plsc.VectorSubcoreMesh(core_axis_name="c", subcore_axis_name="s",
                                  num_cores=NC, num_subcores=NS)
    f32 = jnp.float32
    run = pl.kernel(
        _sc_body,
        out_type=(jax.ShapeDtypeStruct((B, 16), f32),
                  jax.ShapeDtypeStruct((B, 16), f32)),
        mesh=mesh,
        scratch_types=[
            [pltpu.VMEM((K,), jnp.int32) for _ in range(NIDX)],
            pltpu.VMEM((K, D), f32),
            pltpu.VMEM((K, D), f32),
            pltpu.VMEM((K, D), f32),
            pltpu.VMEM((K, 16), f32),
            pltpu.VMEM((K, 16), f32),
            pltpu.SemaphoreType.DMA,
        ],
    )
    return run(in_emb, out_emb, idxb)


def _loss_body(pos_ref, neg_ref, o_ref):
    pos = jnp.sum(pos_ref[...], axis=1) * (1.0 / NCTX)
    neg = jnp.sum(neg_ref[...], axis=1) * (1.0 / NNEG)
    loss = jax.nn.log_sigmoid(pos) + jax.nn.log_sigmoid(-neg)
    o_ref[0, 0] = -jnp.mean(loss)


def _loss_tc(pos_part, neg_part):
    out = pl.pallas_call(
        _loss_body,
        out_shape=jax.ShapeDtypeStruct((1, 1), jnp.float32),
        in_specs=[pl.BlockSpec(memory_space=pltpu.VMEM),
                  pl.BlockSpec(memory_space=pltpu.VMEM)],
        out_specs=pl.BlockSpec(memory_space=pltpu.SMEM),
    )(pos_part, neg_part)
    return out[0, 0]


@jax.jit
def kernel(centers, context, neg_context, in_emb, out_emb):
    centers = centers.astype(jnp.int32)
    context = context.astype(jnp.int32)
    neg_context = neg_context.astype(jnp.int32)
    idx_all = jnp.concatenate(
        [centers[None, :], context.T, neg_context.T], axis=0)
    idxb = idx_all.reshape(NIDX, NW * NCHUNK, K).transpose(1, 0, 2)
    pos, neg = _scores_sc(in_emb, out_emb, idxb)
    return _loss_tc(pos, neg)
